# baseline (speedup 1.0000x reference)
import dataclasses
import functools

import jax
import jax.numpy as jnp
import numpy as np
from jax import lax
from jax.experimental import pallas as pl
from jax.experimental.pallas import tpu as pltpu
from jax.experimental.pallas import tpu_sc as plsc

L = 16
BLOCK = 3200
REP = 16
CHUNKS = (80, 110, 150, 285)

INV_H = np.float32(6.3)
SCALE = np.float32(1.03e-10)

_CP = pltpu.CompilerParams()
_flds = pltpu.CompilerParams.__dataclass_fields__
if "needs_layout_passes" in _flds:
    _CP = dataclasses.replace(_CP, needs_layout_passes=False)
if "use_tc_tiling_on_sc" in _flds:
    _CP = dataclasses.replace(_CP, use_tc_tiling_on_sc=True)


def _interp(t, val_ref, slope_ref, lane):
    ti = t.astype(jnp.int32)
    w = t - ti.astype(jnp.float32)
    y0 = plsc.load_gather(val_ref, [ti, lane])
    dy = plsc.load_gather(slope_ref, [ti, lane])
    return y0 + w * dy


def _make_sc_kernel(n_rows, y_off, mesh):
    @functools.partial(
        pl.kernel,
        out_type=jax.ShapeDtypeStruct((n_rows,), jnp.float32),
        mesh=mesh,
        compiler_params=_CP,
        scratch_types=[
            pltpu.VMEM((64, REP), jnp.float32),
            pltpu.VMEM((64, REP), jnp.float32),
            pltpu.VMEM((64, REP), jnp.float32),
            pltpu.VMEM((64, REP), jnp.float32),
            pltpu.VMEM((64, REP), jnp.float32),
            pltpu.VMEM((64, REP), jnp.float32),
        ],
    )
    def sc_kernel(av_hbm, y_hbm, vco_h, sco_h, vh2_h, sh2_h, vav_h, sav_h,
                  out_hbm, vco_v, sco_v, vh2_v, sh2_v, vav_v, sav_v):
        pltpu.sync_copy(vco_h, vco_v)
        pltpu.sync_copy(sco_h, sco_v)
        pltpu.sync_copy(vh2_h, vh2_v)
        pltpu.sync_copy(sh2_h, sh2_v)
        pltpu.sync_copy(vav_h, vav_v)
        pltpu.sync_copy(sav_h, sav_v)

        def body(av_b, y_b, out_b):
            @pl.loop(0, BLOCK, step=L)
            def _(i):
                lane = lax.iota(jnp.int32, L)
                cols = lane + i
                a = av_b[pl.ds(i, L)]
                yco = plsc.load_gather(
                    y_b, [jnp.full((L,), 5, jnp.int32), cols])
                yh2 = plsc.load_gather(
                    y_b, [jnp.full((L,), 2, jnp.int32), cols])
                s_co = _interp(a * yco * INV_H, vco_v, sco_v, lane)
                s_h2 = _interp(a * yh2 * INV_H, vh2_v, sh2_v, lane)
                s_av = _interp(a * INV_H, vav_v, sav_v, lane)
                out_b[pl.ds(i, L)] = s_co * s_h2 * s_av

        pltpu.emit_pipeline(
            body,
            grid=(n_rows // BLOCK,),
            in_specs=[
                pl.BlockSpec((BLOCK,), lambda i: (i,)),
                pl.BlockSpec((8, BLOCK), lambda i: (0, i + y_off)),
            ],
            out_specs=[pl.BlockSpec((BLOCK,), lambda i: (i,))],
            core_axis_name=("c", "s"),
            dimension_semantics=(pltpu.PARALLEL,),
        )(av_hbm, y_hbm, out_hbm)

    return sc_kernel


def kernel(Av, y_in, x_CO, theta_CO, x_H2, theta_H2, x_Av, theta_Av):
    N = Av.shape[0]
    y_t = lax.transpose(y_in, (1, 0))

    def tables(theta, scale=np.float32(1.0)):
        v = theta * scale
        s = jnp.concatenate([v[1:] - v[:-1], jnp.zeros((1,), jnp.float32)])
        rep = lambda t: jnp.broadcast_to(t[:, None], (t.shape[0], REP))
        return rep(v), rep(s)

    vco, sco = tables(theta_CO)
    vh2, sh2 = tables(theta_H2)
    vav, sav = tables(theta_Av, SCALE)
    tabs = (vco, sco, vh2, sh2, vav, sav)

    mesh = plsc.VectorSubcoreMesh(core_axis_name="c", subcore_axis_name="s")

    outs = []
    off = 0
    for nb in CHUNKS:
        rows = nb * BLOCK
        r0 = off * BLOCK
        av_c = lax.squeeze(lax.slice(Av, (r0, 0), (r0 + rows, 1)), (1,))
        k = _make_sc_kernel(rows, off, mesh)
        o = k(av_c, y_t, *tabs)
        outs.append(lax.reshape(o, (rows, 1)))
        off += nb

    return lax.concatenate(outs, 0)


# speedup vs baseline: 1.1336x; 1.1336x over previous
import dataclasses
import functools

import jax
import jax.numpy as jnp
import numpy as np
from jax import lax
from jax.experimental import pallas as pl
from jax.experimental.pallas import tpu as pltpu
from jax.experimental.pallas import tpu_sc as plsc

L = 16
BLOCK = 3200
REP = 16
SPLIT_BLOCKS = 246

INV_H = np.float32(6.3)
SCALE = np.float32(1.03e-10)

_CP = pltpu.CompilerParams()
_flds = pltpu.CompilerParams.__dataclass_fields__
if "needs_layout_passes" in _flds:
    _CP = dataclasses.replace(_CP, needs_layout_passes=False)
if "use_tc_tiling_on_sc" in _flds:
    _CP = dataclasses.replace(_CP, use_tc_tiling_on_sc=True)


def _interp(t, val_ref, slope_ref, lane):
    ti = t.astype(jnp.int32)
    w = t - ti.astype(jnp.float32)
    y0 = plsc.load_gather(val_ref, [ti, lane])
    dy = plsc.load_gather(slope_ref, [ti, lane])
    return y0 + w * dy


def _make_sc_kernel(n_rows, y_off, mesh):
    @functools.partial(
        pl.kernel,
        out_type=jax.ShapeDtypeStruct((n_rows,), jnp.float32),
        mesh=mesh,
        compiler_params=_CP,
        scratch_types=[
            pltpu.VMEM((64, REP), jnp.float32),
            pltpu.VMEM((64, REP), jnp.float32),
            pltpu.VMEM((64, REP), jnp.float32),
            pltpu.VMEM((64, REP), jnp.float32),
            pltpu.VMEM((64, REP), jnp.float32),
            pltpu.VMEM((64, REP), jnp.float32),
        ],
    )
    def sc_kernel(av_hbm, y_hbm, vco_h, sco_h, vh2_h, sh2_h, vav_h, sav_h,
                  out_hbm, vco_v, sco_v, vh2_v, sh2_v, vav_v, sav_v):
        pltpu.sync_copy(vco_h, vco_v)
        pltpu.sync_copy(sco_h, sco_v)
        pltpu.sync_copy(vh2_h, vh2_v)
        pltpu.sync_copy(sh2_h, sh2_v)
        pltpu.sync_copy(vav_h, vav_v)
        pltpu.sync_copy(sav_h, sav_v)

        def body(av_b, y_b, out_b):
            @pl.loop(0, BLOCK, step=L)
            def _(i):
                lane = lax.iota(jnp.int32, L)
                cols = lane + i
                a = av_b[pl.ds(i, L)]
                yco = plsc.load_gather(
                    y_b, [jnp.full((L,), 5, jnp.int32), cols])
                yh2 = plsc.load_gather(
                    y_b, [jnp.full((L,), 2, jnp.int32), cols])
                s_co = _interp(a * yco * INV_H, vco_v, sco_v, lane)
                s_h2 = _interp(a * yh2 * INV_H, vh2_v, sh2_v, lane)
                s_av = _interp(a * INV_H, vav_v, sav_v, lane)
                out_b[pl.ds(i, L)] = s_co * s_h2 * s_av

        pltpu.emit_pipeline(
            body,
            grid=(n_rows // BLOCK,),
            in_specs=[
                pl.BlockSpec((BLOCK,), lambda i: (i,)),
                pl.BlockSpec((8, BLOCK), lambda i: (0, i + y_off)),
            ],
            out_specs=[pl.BlockSpec((BLOCK,), lambda i: (i,))],
            core_axis_name=("c", "s"),
            dimension_semantics=(pltpu.PARALLEL,),
        )(av_hbm, y_hbm, out_hbm)

    return sc_kernel


def kernel(Av, y_in, x_CO, theta_CO, x_H2, theta_H2, x_Av, theta_Av):
    N = Av.shape[0]
    S = SPLIT_BLOCKS * BLOCK
    y_t = lax.transpose(y_in, (1, 0))
    av1 = lax.squeeze(lax.slice(Av, (0, 0), (S, 1)), (1,))
    av2 = lax.squeeze(lax.slice(Av, (S, 0), (N, 1)), (1,))

    def tables(theta, scale=np.float32(1.0)):
        v = theta * scale
        s = jnp.concatenate([v[1:] - v[:-1], jnp.zeros((1,), jnp.float32)])
        rep = lambda t: jnp.broadcast_to(t[:, None], (t.shape[0], REP))
        return rep(v), rep(s)

    vco, sco = tables(theta_CO)
    vh2, sh2 = tables(theta_H2)
    vav, sav = tables(theta_Av, SCALE)

    mesh = plsc.VectorSubcoreMesh(core_axis_name="c", subcore_axis_name="s")
    k1 = _make_sc_kernel(S, 0, mesh)
    k2 = _make_sc_kernel(N - S, SPLIT_BLOCKS, mesh)

    o1 = k1(av1, y_t, vco, sco, vh2, sh2, vav, sav)
    o2 = k2(av2, y_t, vco, sco, vh2, sh2, vav, sav)
    o1r = lax.reshape(o1, (S, 1))
    o2r = lax.reshape(o2, (N - S, 1))
    return lax.concatenate([o1r, o2r], 0)


# speedup vs baseline: 1.2804x; 1.1296x over previous
import dataclasses
import functools

import jax
import jax.numpy as jnp
import numpy as np
from jax import lax
from jax.experimental import pallas as pl
from jax.experimental.pallas import tpu as pltpu
from jax.experimental.pallas import tpu_sc as plsc

L = 16
BLOCK = 3200
REP = 16
SPLIT_BLOCKS = 246

INV_H = np.float32(6.3)
SCALE = np.float32(1.03e-10)

_CP = pltpu.CompilerParams()
_flds = pltpu.CompilerParams.__dataclass_fields__
if "needs_layout_passes" in _flds:
    _CP = dataclasses.replace(_CP, needs_layout_passes=False)
if "use_tc_tiling_on_sc" in _flds:
    _CP = dataclasses.replace(_CP, use_tc_tiling_on_sc=True)


def _interp(t, val_ref, slope_ref, lane):
    ti = t.astype(jnp.int32)
    w = t - ti.astype(jnp.float32)
    y0 = plsc.load_gather(val_ref, [ti, lane])
    dy = plsc.load_gather(slope_ref, [ti, lane])
    return y0 + w * dy


def _make_sc_kernel(n_rows, y_off, mesh):
    @functools.partial(
        pl.kernel,
        out_type=jax.ShapeDtypeStruct((n_rows,), jnp.float32),
        mesh=mesh,
        compiler_params=_CP,
        scratch_types=[
            pltpu.VMEM((64, REP), jnp.float32),
            pltpu.VMEM((64, REP), jnp.float32),
            pltpu.VMEM((64, REP), jnp.float32),
            pltpu.VMEM((64, REP), jnp.float32),
            pltpu.VMEM((64, REP), jnp.float32),
            pltpu.VMEM((64, REP), jnp.float32),
        ],
    )
    def sc_kernel(av_hbm, y_hbm, vco_h, sco_h, vh2_h, sh2_h, vav_h, sav_h,
                  out_hbm, vco_v, sco_v, vh2_v, sh2_v, vav_v, sav_v):
        pltpu.sync_copy(vco_h, vco_v)
        pltpu.sync_copy(sco_h, sco_v)
        pltpu.sync_copy(vh2_h, vh2_v)
        pltpu.sync_copy(sh2_h, sh2_v)
        pltpu.sync_copy(vav_h, vav_v)
        pltpu.sync_copy(sav_h, sav_v)

        def body(av_b, y_b, out_b):
            @pl.loop(0, BLOCK, step=L)
            def _(i):
                lane = lax.iota(jnp.int32, L)
                cols = lane + i
                a = av_b[pl.ds(i, L)]
                yco = plsc.load_gather(
                    y_b, [jnp.full((L,), 5, jnp.int32), cols])
                yh2 = plsc.load_gather(
                    y_b, [jnp.full((L,), 2, jnp.int32), cols])
                s_co = _interp(a * yco * INV_H, vco_v, sco_v, lane)
                s_h2 = _interp(a * yh2 * INV_H, vh2_v, sh2_v, lane)
                s_av = _interp(a * INV_H, vav_v, sav_v, lane)
                out_b[pl.ds(i, L)] = s_co * s_h2 * s_av

        pltpu.emit_pipeline(
            body,
            grid=(n_rows // BLOCK,),
            in_specs=[
                pl.BlockSpec((BLOCK,), lambda i: (i,)),
                pl.BlockSpec((8, BLOCK), lambda i: (0, i + y_off)),
            ],
            out_specs=[pl.BlockSpec((BLOCK,), lambda i: (i,))],
            core_axis_name=("c", "s"),
            dimension_semantics=(pltpu.PARALLEL,),
        )(av_hbm, y_hbm, out_hbm)

    return sc_kernel


def kernel(Av, y_in, x_CO, theta_CO, x_H2, theta_H2, x_Av, theta_Av):
    N = Av.shape[0]
    S = SPLIT_BLOCKS * BLOCK
    y_t = lax.transpose(y_in, (1, 0))
    av1 = lax.squeeze(lax.slice(Av, (0, 0), (S, 1)), (1,))
    av2 = lax.squeeze(lax.slice(Av, (S, 0), (N, 1)), (1,))

    def tables(theta, scale=np.float32(1.0)):
        v = theta * scale
        s = jnp.concatenate([v[1:] - v[:-1], jnp.zeros((1,), jnp.float32)])
        rep = lambda t: jnp.broadcast_to(t[:, None], (t.shape[0], REP))
        return rep(v), rep(s)

    vco, sco = tables(theta_CO)
    vh2, sh2 = tables(theta_H2)
    vav, sav = tables(theta_Av, SCALE)

    mesh = plsc.VectorSubcoreMesh(core_axis_name="c", subcore_axis_name="s")
    k1 = _make_sc_kernel(S, 0, mesh)
    k2 = _make_sc_kernel(N - S, SPLIT_BLOCKS, mesh)

    o1 = k1(av1, y_t, vco, sco, vh2, sh2, vav, sav)
    o2 = k2(av2, y_t, vco, sco, vh2, sh2, vav, sav)
    out = jnp.concatenate([o1, o2])
    return out.reshape(N, 1)
